# baseline (speedup 1.0000x reference)
import functools

import jax
import jax.numpy as jnp
from jax import lax
from jax.experimental import pallas as pl
from jax.experimental.pallas import tpu as pltpu
from jax.experimental.pallas import tpu_sc as plsc

_NC = 2
_NS = 16
_NW = _NC * _NS


def _proj_body(a_ref, b_ref, o_ref):
    o_ref[...] = jnp.dot(a_ref[...], b_ref[...],
                         preferred_element_type=jnp.float32)


def _project_table(emb_weight, proj_Wt):
    V, D = emb_weight.shape
    H = proj_Wt.shape[1]
    BM = 1000
    assert V % BM == 0
    return pl.pallas_call(
        _proj_body,
        grid=(V // BM,),
        in_specs=[
            pl.BlockSpec((BM, D), lambda i: (i, 0)),
            pl.BlockSpec((D, H), lambda i: (0, 0)),
        ],
        out_specs=pl.BlockSpec((BM, H), lambda i: (i, 0)),
        out_shape=jax.ShapeDtypeStruct((V, H), jnp.float32),
    )(emb_weight, proj_Wt)


def _fin_body(s_ref, l_ref, b_ref, o_ref):
    inv = 1.0 / l_ref[...].astype(jnp.float32)
    o_ref[...] = s_ref[...] * inv + b_ref[...]


def _finalize(sums, length, proj_b):
    B, H = sums.shape
    return pl.pallas_call(
        _fin_body,
        in_specs=[
            pl.BlockSpec((B, H), lambda: (0, 0)),
            pl.BlockSpec((B, 1), lambda: (0, 0)),
            pl.BlockSpec((1, H), lambda: (0, 0)),
        ],
        out_specs=pl.BlockSpec((B, H), lambda: (0, 0)),
        out_shape=jax.ShapeDtypeStruct((B, H), jnp.float32),
    )(sums, length.reshape(B, 1), proj_b.reshape(1, H))


def _make_sc_pool(B, H, CL, NCH):
    RB = B // _NW
    HV = H // 16

    def _accum(buf):
        def jbody(j, carry):
            return tuple(carry[k] + buf[j, pl.ds(16 * k, 16)]
                         for k in range(HV))
        init = tuple(jnp.zeros((16,), jnp.float32) for _ in range(HV))
        return lax.fori_loop(0, CL, jbody, init, unroll=4)

    NBUF = 4

    @functools.partial(
        pl.kernel,
        mesh=plsc.VectorSubcoreMesh(core_axis_name="c", subcore_axis_name="s"),
        out_type=jax.ShapeDtypeStruct((B, H), jnp.float32),
        scratch_types=(
            [pltpu.VMEM((RB, NCH, CL), jnp.int32)]
            + [pltpu.VMEM((CL, H), jnp.float32) for _ in range(NBUF)]
            + [pltpu.VMEM((RB, H), jnp.float32)]
            + [pltpu.SemaphoreType.DMA for _ in range(NBUF)]
        ),
    )
    def sc_pool(xp_hbm, table_hbm, sums_hbm, idx_v, b0, b1, b2, b3, out_v,
                s0, s1, s2, s3):
        bufs = (b0, b1, b2, b3)
        sems = (s0, s1, s2, s3)
        wid = lax.axis_index("s") * _NC + lax.axis_index("c")
        base = wid * RB
        pltpu.sync_copy(xp_hbm.at[pl.ds(base, RB)], idx_v)
        for i in range(NBUF):
            pltpu.make_async_copy(
                table_hbm.at[idx_v.at[i >> 1, i & 1]], bufs[i],
                sems[i]).start()

        def group_body(g, carry):
            r0 = 2 * g
            accs = []
            for i in range(NBUF):
                row = r0 + (i >> 1)
                half = i & 1
                pltpu.make_async_copy(
                    table_hbm.at[idx_v.at[row, half]], bufs[i],
                    sems[i]).wait()
                accs.append(_accum(bufs[i]))
                nrow = row + 2

                @pl.when(nrow < RB)
                def _():
                    pltpu.make_async_copy(
                        table_hbm.at[idx_v.at[nrow, half]], bufs[i],
                        sems[i]).start()

            for k in range(HV):
                out_v[r0, pl.ds(16 * k, 16)] = accs[0][k] + accs[1][k]
                out_v[r0 + 1, pl.ds(16 * k, 16)] = accs[2][k] + accs[3][k]
            return carry

        lax.fori_loop(0, RB // 2, group_body, 0)
        pltpu.sync_copy(out_v, sums_hbm.at[pl.ds(base, RB)])

    return sc_pool


def kernel(x, length, emb_weight, proj_W, proj_b):
    B, L = x.shape
    H, D = proj_W.shape
    CL = 104
    NCH = 2
    pad = NCH * CL - L
    xi = x.astype(jnp.int32)
    xp = jnp.concatenate(
        [xi, jnp.full((B, pad), 1, jnp.int32)], axis=1).reshape(B, NCH, CL)

    emb_proj = _project_table(emb_weight, proj_W.T)
    sums = _make_sc_pool(B, H, CL, NCH)(xp, emb_proj)
    return _finalize(sums, length, proj_b)


# speedup vs baseline: 1.1565x; 1.1565x over previous
import functools

import jax
import jax.numpy as jnp
import numpy as np
from jax import lax
from jax.experimental import pallas as pl
from jax.experimental.pallas import tpu as pltpu
from jax.experimental.pallas import tpu_sc as plsc

_NC = 2
_NS = 16
_NW = _NC * _NS


def _proj_body(a_ref, b_ref, o_ref):
    o_ref[...] = jnp.dot(a_ref[...], b_ref[...],
                         preferred_element_type=jnp.float32
                         ).astype(jnp.bfloat16)


def _project_table(emb_weight, proj_Wt):
    V, D = emb_weight.shape
    H = proj_Wt.shape[1]
    BM = 1000
    assert V % BM == 0
    return pl.pallas_call(
        _proj_body,
        grid=(V // BM,),
        in_specs=[
            pl.BlockSpec((BM, D), lambda i: (i, 0)),
            pl.BlockSpec((D, H), lambda i: (0, 0)),
        ],
        out_specs=pl.BlockSpec((BM, H), lambda i: (i, 0)),
        out_shape=jax.ShapeDtypeStruct((V, H), jnp.bfloat16),
    )(emb_weight, proj_Wt)


def _fin_body(s_ref, l_ref, b_ref, o_ref):
    inv = 1.0 / l_ref[...].astype(jnp.float32)
    o_ref[...] = s_ref[...] * inv + b_ref[...]


def _finalize(sums, length, proj_b):
    B, H = sums.shape
    return pl.pallas_call(
        _fin_body,
        in_specs=[
            pl.BlockSpec((B, H), lambda: (0, 0)),
            pl.BlockSpec((B, 1), lambda: (0, 0)),
            pl.BlockSpec((1, H), lambda: (0, 0)),
        ],
        out_specs=pl.BlockSpec((B, H), lambda: (0, 0)),
        out_shape=jax.ShapeDtypeStruct((B, H), jnp.float32),
    )(sums, length.reshape(B, 1), proj_b.reshape(1, H))


def _unpack_perm(H):
    perm = np.empty((H,), dtype=np.int32)
    for k in range(H // 32):
        for i in range(16):
            perm[32 * k + 2 * i] = 16 * k + i
            perm[32 * k + 2 * i + 1] = H // 2 + 16 * k + i
    return perm


def _make_sc_pool(B, H, CL, NCH):
    RB = B // _NW
    HP = H // 32
    HW = H // 2
    NBUF = 4

    def _accum(buf):
        hi_mask = jnp.int32(-65536)

        def jbody(j, carry):
            acc = list(carry)
            for k in range(HP):
                w = buf[j, pl.ds(16 * k, 16)]
                a = plsc.bitcast(w << 16, jnp.float32)
                b = plsc.bitcast(w & hi_mask, jnp.float32)
                acc[k] = acc[k] + a
                acc[HP + k] = acc[HP + k] + b
            return tuple(acc)
        init = tuple(jnp.zeros((16,), jnp.float32) for _ in range(2 * HP))
        return lax.fori_loop(0, CL, jbody, init, unroll=4)

    @functools.partial(
        pl.kernel,
        mesh=plsc.VectorSubcoreMesh(core_axis_name="c", subcore_axis_name="s"),
        out_type=jax.ShapeDtypeStruct((B, H), jnp.float32),
        compiler_params=pltpu.CompilerParams(
            needs_layout_passes=False, use_tc_tiling_on_sc=False),
        scratch_types=(
            [pltpu.VMEM((RB, NCH, CL), jnp.int32)]
            + [pltpu.VMEM((CL, HW), jnp.int32) for _ in range(NBUF)]
            + [pltpu.VMEM((RB, H), jnp.float32)]
            + [pltpu.SemaphoreType.DMA for _ in range(NBUF)]
        ),
    )
    def sc_pool(xp_hbm, table_hbm, sums_hbm, idx_v, b0, b1, b2, b3, out_v,
                s0, s1, s2, s3):
        bufs = (b0, b1, b2, b3)
        sems = (s0, s1, s2, s3)
        wid = lax.axis_index("s") * _NC + lax.axis_index("c")
        base = wid * RB
        pltpu.sync_copy(xp_hbm.at[pl.ds(base, RB)], idx_v)
        for i in range(NBUF):
            pltpu.make_async_copy(
                table_hbm.at[idx_v.at[i >> 1, i & 1]], bufs[i],
                sems[i]).start()

        def group_body(g, carry):
            r0 = 2 * g
            accs = []
            for i in range(NBUF):
                row = r0 + (i >> 1)
                half = i & 1
                pltpu.make_async_copy(
                    table_hbm.at[idx_v.at[row, half]], bufs[i],
                    sems[i]).wait()
                accs.append(_accum(bufs[i]))
                nrow = row + 2

                @pl.when(nrow < RB)
                def _():
                    pltpu.make_async_copy(
                        table_hbm.at[idx_v.at[nrow, half]], bufs[i],
                        sems[i]).start()

            for k in range(2 * HP):
                out_v[r0, pl.ds(16 * k, 16)] = accs[0][k] + accs[1][k]
                out_v[r0 + 1, pl.ds(16 * k, 16)] = accs[2][k] + accs[3][k]
            return carry

        lax.fori_loop(0, RB // 2, group_body, 0)
        pltpu.sync_copy(out_v, sums_hbm.at[pl.ds(base, RB)])

    return sc_pool


def kernel(x, length, emb_weight, proj_W, proj_b):
    B, L = x.shape
    H, D = proj_W.shape
    CL = 104
    NCH = 2
    pad = NCH * CL - L
    xi = x.astype(jnp.int32)
    xp = jnp.concatenate(
        [xi, jnp.full((B, pad), 1, jnp.int32)], axis=1).reshape(B, NCH, CL)

    proj_Wt = proj_W.T[:, _unpack_perm(H)]
    emb_proj = _project_table(emb_weight, proj_Wt)
    V = emb_proj.shape[0]
    table_i32 = jax.lax.bitcast_convert_type(
        emb_proj.reshape(V, H // 2, 2), jnp.int32)
    sums = _make_sc_pool(B, H, CL, NCH)(xp, table_i32)
    return _finalize(sums, length, proj_b)


# speedup vs baseline: 4.1071x; 3.5514x over previous
import functools

import jax
import jax.numpy as jnp
from jax import lax
from jax.experimental import pallas as pl
from jax.experimental.pallas import tpu as pltpu
from jax.experimental.pallas import tpu_sc as plsc

_NC = 2
_NS = 16
_NW = _NC * _NS


def _proj_body(a_ref, blo_ref, bhi_ref, o_ref):
    mlo = jnp.dot(a_ref[...], blo_ref[...],
                  preferred_element_type=jnp.float32)
    mhi = jnp.dot(a_ref[...], bhi_ref[...],
                  preferred_element_type=jnp.float32)
    lo = mlo.astype(jnp.bfloat16).astype(jnp.float32)
    hi = mhi.astype(jnp.bfloat16).astype(jnp.float32)
    lo_u = jax.lax.bitcast_convert_type(lo, jnp.uint32) >> 16
    hi_u = jax.lax.bitcast_convert_type(hi, jnp.uint32) & jnp.uint32(
        0xFFFF0000)
    o_ref[...] = jax.lax.bitcast_convert_type(lo_u | hi_u, jnp.int32)


def _project_table(emb_weight, proj_Wt):
    V, D = emb_weight.shape
    H = proj_Wt.shape[1]
    HW = H // 2
    BM = 1000
    assert V % BM == 0
    return pl.pallas_call(
        _proj_body,
        grid=(V // BM,),
        in_specs=[
            pl.BlockSpec((BM, D), lambda i: (i, 0)),
            pl.BlockSpec((D, HW), lambda i: (0, 0)),
            pl.BlockSpec((D, HW), lambda i: (0, 0)),
        ],
        out_specs=pl.BlockSpec((BM, HW), lambda i: (i, 0)),
        out_shape=jax.ShapeDtypeStruct((V, HW), jnp.int32),
    )(emb_weight, proj_Wt[:, :HW], proj_Wt[:, HW:])


def _fin_body(s_ref, l_ref, b_ref, o_ref):
    inv = 1.0 / l_ref[...].astype(jnp.float32)
    o_ref[...] = s_ref[...] * inv + b_ref[...]


def _finalize(sums, length, proj_b):
    B, H = sums.shape
    return pl.pallas_call(
        _fin_body,
        in_specs=[
            pl.BlockSpec((B, H), lambda: (0, 0)),
            pl.BlockSpec((B, 1), lambda: (0, 0)),
            pl.BlockSpec((1, H), lambda: (0, 0)),
        ],
        out_specs=pl.BlockSpec((B, H), lambda: (0, 0)),
        out_shape=jax.ShapeDtypeStruct((B, H), jnp.float32),
    )(sums, length.reshape(B, 1), proj_b.reshape(1, H))


def _make_sc_pool(B, L, H):
    RB = B // _NW
    HP = H // 32
    HW = H // 2
    CL0 = 104
    CL1 = L - CL0

    def _accum(buf, n):
        hi_mask = jnp.int32(-65536)

        def jbody(j, carry):
            acc = list(carry)
            for k in range(HP):
                w = buf[j, pl.ds(16 * k, 16)]
                a = plsc.bitcast(w << 16, jnp.float32)
                b = plsc.bitcast(w & hi_mask, jnp.float32)
                acc[k] = acc[k] + a
                acc[HP + k] = acc[HP + k] + b
            return tuple(acc)
        init = tuple(jnp.zeros((16,), jnp.float32) for _ in range(2 * HP))
        return lax.fori_loop(0, n, jbody, init, unroll=4)

    def _idx(idx_v, row, half):
        if half == 0:
            return idx_v.at[row, pl.ds(0, CL0)]
        return idx_v.at[row, pl.ds(CL0, CL1)]

    @functools.partial(
        pl.kernel,
        mesh=plsc.VectorSubcoreMesh(core_axis_name="c", subcore_axis_name="s"),
        out_type=jax.ShapeDtypeStruct((B, H), jnp.float32),
        compiler_params=pltpu.CompilerParams(
            needs_layout_passes=False, use_tc_tiling_on_sc=False),
        scratch_types=(
            [pltpu.VMEM((RB, L), jnp.int32)]
            + [pltpu.VMEM((CL0, HW), jnp.int32),
               pltpu.VMEM((CL1, HW), jnp.int32),
               pltpu.VMEM((CL0, HW), jnp.int32),
               pltpu.VMEM((CL1, HW), jnp.int32)]
            + [pltpu.VMEM((RB, H), jnp.float32)]
            + [pltpu.SemaphoreType.DMA for _ in range(4)]
        ),
    )
    def sc_pool(x_hbm, table_hbm, sums_hbm, idx_v, b0, b1, b2, b3, out_v,
                s0, s1, s2, s3):
        bufs = (b0, b1, b2, b3)
        sems = (s0, s1, s2, s3)
        lens = (CL0, CL1, CL0, CL1)
        wid = lax.axis_index("s") * _NC + lax.axis_index("c")
        base = wid * RB
        pltpu.sync_copy(x_hbm.at[pl.ds(base, RB)], idx_v)
        for i in range(4):
            pltpu.make_async_copy(
                table_hbm.at[_idx(idx_v, i >> 1, i & 1)], bufs[i],
                sems[i]).start()

        def group_body(g, carry):
            r0 = 2 * g
            accs = []
            for i in range(4):
                row = r0 + (i >> 1)
                half = i & 1
                pltpu.make_async_copy(
                    table_hbm.at[_idx(idx_v, row, half)], bufs[i],
                    sems[i]).wait()
                accs.append(_accum(bufs[i], lens[i]))
                nrow = row + 2

                @pl.when(nrow < RB)
                def _():
                    pltpu.make_async_copy(
                        table_hbm.at[_idx(idx_v, nrow, half)], bufs[i],
                        sems[i]).start()

            for k in range(2 * HP):
                out_v[r0, pl.ds(16 * k, 16)] = accs[0][k] + accs[1][k]
                out_v[r0 + 1, pl.ds(16 * k, 16)] = accs[2][k] + accs[3][k]
            return carry

        lax.fori_loop(0, RB // 2, group_body, 0)
        pltpu.sync_copy(out_v, sums_hbm.at[pl.ds(base, RB)])

    return sc_pool


def kernel(x, length, emb_weight, proj_W, proj_b):
    B, L = x.shape
    H, D = proj_W.shape
    xi = x.astype(jnp.int32)
    table = _project_table(emb_weight, proj_W.T)
    sums = _make_sc_pool(B, L, H)(xi, table)
    return _finalize(sums, length, proj_b)


# speedup vs baseline: 5.0430x; 1.2279x over previous
import functools

import jax
import jax.numpy as jnp
from jax import lax
from jax.experimental import pallas as pl
from jax.experimental.pallas import tpu as pltpu
from jax.experimental.pallas import tpu_sc as plsc

_NC = 2
_NS = 16
_NW = _NC * _NS


def _proj_body(a_ref, blo_ref, bhi_ref, o_ref):
    mlo = jnp.dot(a_ref[...], blo_ref[...],
                  preferred_element_type=jnp.float32)
    mhi = jnp.dot(a_ref[...], bhi_ref[...],
                  preferred_element_type=jnp.float32)
    lo = mlo.astype(jnp.bfloat16).astype(jnp.float32)
    hi = mhi.astype(jnp.bfloat16).astype(jnp.float32)
    lo_u = jax.lax.bitcast_convert_type(lo, jnp.uint32) >> 16
    hi_u = jax.lax.bitcast_convert_type(hi, jnp.uint32) & jnp.uint32(
        0xFFFF0000)
    w = jax.lax.bitcast_convert_type(lo_u | hi_u, jnp.int32)
    o_ref[...] = jnp.concatenate([w, jnp.zeros_like(w)], axis=1)


def _project_table(emb_weight, proj_Wt):
    V, D = emb_weight.shape
    H = proj_Wt.shape[1]
    HW = H // 2
    BM = 2000
    assert V % BM == 0 and (BM // 2) % 8 == 0
    return pl.pallas_call(
        _proj_body,
        grid=(V // BM,),
        in_specs=[
            pl.BlockSpec((BM, D), lambda i: (i, 0)),
            pl.BlockSpec((D, HW), lambda i: (0, 0)),
            pl.BlockSpec((D, HW), lambda i: (0, 0)),
        ],
        out_specs=pl.BlockSpec((BM, 2 * HW), lambda i: (i, 0)),
        out_shape=jax.ShapeDtypeStruct((V, 2 * HW), jnp.int32),
    )(emb_weight, proj_Wt[:, :HW], proj_Wt[:, HW:]).reshape(2 * V, HW)


def _fin_body(s_ref, l_ref, b_ref, o_ref):
    inv = 1.0 / l_ref[...].astype(jnp.float32)
    o_ref[...] = s_ref[...] * inv + b_ref[...]


def _finalize(sums, length, proj_b):
    B, H = sums.shape
    return pl.pallas_call(
        _fin_body,
        in_specs=[
            pl.BlockSpec((B, H), lambda: (0, 0)),
            pl.BlockSpec((B, 1), lambda: (0, 0)),
            pl.BlockSpec((1, H), lambda: (0, 0)),
        ],
        out_specs=pl.BlockSpec((B, H), lambda: (0, 0)),
        out_shape=jax.ShapeDtypeStruct((B, H), jnp.float32),
    )(sums, length.reshape(B, 1), proj_b.reshape(1, H))


def _make_sc_pool(B, L, H):
    RB = B // _NW
    HP = H // 32
    HW = H // 2
    CL0 = 104
    CL1 = L - CL0

    def _accum(buf, n):
        hi_mask = jnp.int32(-65536)

        def jbody(j, carry):
            acc = list(carry)
            for k in range(HP):
                w = buf[j, pl.ds(16 * k, 16)]
                a = plsc.bitcast(w << 16, jnp.float32)
                b = plsc.bitcast(w & hi_mask, jnp.float32)
                acc[k] = acc[k] + a
                acc[HP + k] = acc[HP + k] + b
            return tuple(acc)
        init = tuple(jnp.zeros((16,), jnp.float32) for _ in range(2 * HP))
        return lax.fori_loop(0, n, jbody, init, unroll=4)

    def _idx(idx_v, row, half):
        if half == 0:
            return idx_v.at[row, pl.ds(0, CL0)]
        return idx_v.at[row, pl.ds(CL0, CL1)]

    @functools.partial(
        pl.kernel,
        mesh=plsc.VectorSubcoreMesh(core_axis_name="c", subcore_axis_name="s"),
        out_type=jax.ShapeDtypeStruct((B, H), jnp.float32),
        compiler_params=pltpu.CompilerParams(
            needs_layout_passes=False, use_tc_tiling_on_sc=False),
        scratch_types=(
            [pltpu.VMEM((RB, L), jnp.int32)]
            + [pltpu.VMEM((CL0, HW), jnp.int32),
               pltpu.VMEM((CL1, HW), jnp.int32),
               pltpu.VMEM((CL0, HW), jnp.int32),
               pltpu.VMEM((CL1, HW), jnp.int32)]
            + [pltpu.VMEM((RB, H), jnp.float32)]
            + [pltpu.SemaphoreType.DMA for _ in range(4)]
        ),
    )
    def sc_pool(x_hbm, table_hbm, sums_hbm, idx_v, b0, b1, b2, b3, out_v,
                s0, s1, s2, s3):
        bufs = (b0, b1, b2, b3)
        sems = (s0, s1, s2, s3)
        lens = (CL0, CL1, CL0, CL1)
        wid = lax.axis_index("s") * _NC + lax.axis_index("c")
        base = wid * RB
        pltpu.sync_copy(x_hbm.at[pl.ds(base, RB)], idx_v)
        for i in range(4):
            pltpu.make_async_copy(
                table_hbm.at[_idx(idx_v, i >> 1, i & 1)], bufs[i],
                sems[i]).start()

        def group_body(g, carry):
            r0 = 2 * g
            accs = []
            for i in range(4):
                row = r0 + (i >> 1)
                half = i & 1
                pltpu.make_async_copy(
                    table_hbm.at[_idx(idx_v, row, half)], bufs[i],
                    sems[i]).wait()
                accs.append(_accum(bufs[i], lens[i]))
                nrow = row + 2

                @pl.when(nrow < RB)
                def _():
                    pltpu.make_async_copy(
                        table_hbm.at[_idx(idx_v, nrow, half)], bufs[i],
                        sems[i]).start()

            for k in range(2 * HP):
                out_v[r0, pl.ds(16 * k, 16)] = accs[0][k] + accs[1][k]
                out_v[r0 + 1, pl.ds(16 * k, 16)] = accs[2][k] + accs[3][k]
            return carry

        lax.fori_loop(0, RB // 2, group_body, 0)
        pltpu.sync_copy(out_v, sums_hbm.at[pl.ds(base, RB)])

    return sc_pool


def kernel(x, length, emb_weight, proj_W, proj_b):
    B, L = x.shape
    H, D = proj_W.shape
    xi = x.astype(jnp.int32) * 2
    table = _project_table(emb_weight, proj_W.T)
    sums = _make_sc_pool(B, L, H)(xi, table)
    return _finalize(sums, length, proj_b)


# speedup vs baseline: 8.7590x; 1.7369x over previous
import functools

import jax
import jax.numpy as jnp
from jax import lax
from jax.experimental import pallas as pl
from jax.experimental.pallas import tpu as pltpu
from jax.experimental.pallas import tpu_sc as plsc

_NC = 2
_NS = 16
_NW = _NC * _NS


def _proj_body(a_ref, blo_ref, bhi_ref, o_ref):
    dn = (((0,), (0,)), ((), ()))
    mlo = lax.dot_general(a_ref[...], blo_ref[...], dn,
                          preferred_element_type=jnp.float32)
    mhi = lax.dot_general(a_ref[...], bhi_ref[...], dn,
                          preferred_element_type=jnp.float32)
    lo = mlo.astype(jnp.bfloat16).astype(jnp.float32)
    hi = mhi.astype(jnp.bfloat16).astype(jnp.float32)
    lo_u = jax.lax.bitcast_convert_type(lo, jnp.uint32) >> 16
    hi_u = jax.lax.bitcast_convert_type(hi, jnp.uint32) & jnp.uint32(
        0xFFFF0000)
    w = jax.lax.bitcast_convert_type(lo_u | hi_u, jnp.int32)
    o_ref[...] = jnp.concatenate([w, jnp.zeros_like(w)], axis=1)


def _project_table(emb_weight, proj_Wt):
    V, D = emb_weight.shape
    H = proj_Wt.shape[1]
    HW = H // 2
    BM = 6400
    grid = (V + BM - 1) // BM
    return pl.pallas_call(
        _proj_body,
        grid=(grid,),
        in_specs=[
            pl.BlockSpec((D, BM), lambda i: (0, i)),
            pl.BlockSpec((D, HW), lambda i: (0, 0)),
            pl.BlockSpec((D, HW), lambda i: (0, 0)),
        ],
        out_specs=pl.BlockSpec((BM, 2 * HW), lambda i: (i, 0)),
        out_shape=jax.ShapeDtypeStruct((V, 2 * HW), jnp.int32),
    )(emb_weight.T, proj_Wt[:, :HW], proj_Wt[:, HW:]).reshape(2 * V, HW)


def _fin_body(s_ref, l_ref, b_ref, o_ref):
    inv = 1.0 / l_ref[...].astype(jnp.float32)
    o_ref[...] = s_ref[...] * inv + b_ref[...]


def _finalize(sums, length, proj_b):
    B, H = sums.shape
    return pl.pallas_call(
        _fin_body,
        in_specs=[
            pl.BlockSpec((B, H), lambda: (0, 0)),
            pl.BlockSpec((B, 1), lambda: (0, 0)),
            pl.BlockSpec((1, H), lambda: (0, 0)),
        ],
        out_specs=pl.BlockSpec((B, H), lambda: (0, 0)),
        out_shape=jax.ShapeDtypeStruct((B, H), jnp.float32),
    )(sums, length.reshape(B, 1), proj_b.reshape(1, H))


def _make_sc_pool(B, L, H):
    RB = B // _NW
    HP = H // 32
    HW = H // 2
    CL0 = 104
    CL1 = L - CL0

    def _accum(buf, n):
        hi_mask = jnp.int32(-65536)

        def jbody(j, carry):
            acc = list(carry)
            for k in range(HP):
                w = buf[j, pl.ds(16 * k, 16)]
                a = plsc.bitcast(w << 16, jnp.float32)
                b = plsc.bitcast(w & hi_mask, jnp.float32)
                acc[k] = acc[k] + a
                acc[HP + k] = acc[HP + k] + b
            return tuple(acc)
        init = tuple(jnp.zeros((16,), jnp.float32) for _ in range(2 * HP))
        return lax.fori_loop(0, n, jbody, init, unroll=4)

    def _idx(idx_v, row, half):
        if half == 0:
            return idx_v.at[row, pl.ds(0, CL0)]
        return idx_v.at[row, pl.ds(CL0, CL1)]

    @functools.partial(
        pl.kernel,
        mesh=plsc.VectorSubcoreMesh(core_axis_name="c", subcore_axis_name="s"),
        out_type=jax.ShapeDtypeStruct((B, H), jnp.float32),
        compiler_params=pltpu.CompilerParams(
            needs_layout_passes=False, use_tc_tiling_on_sc=False),
        scratch_types=(
            [pltpu.VMEM((RB, L), jnp.int32)]
            + [pltpu.VMEM((CL0, HW), jnp.int32),
               pltpu.VMEM((CL1, HW), jnp.int32),
               pltpu.VMEM((CL0, HW), jnp.int32),
               pltpu.VMEM((CL1, HW), jnp.int32)]
            + [pltpu.VMEM((RB, H), jnp.float32)]
            + [pltpu.SemaphoreType.DMA for _ in range(4)]
        ),
    )
    def sc_pool(x_hbm, table_hbm, sums_hbm, idx_v, b0, b1, b2, b3, out_v,
                s0, s1, s2, s3):
        bufs = (b0, b1, b2, b3)
        sems = (s0, s1, s2, s3)
        lens = (CL0, CL1, CL0, CL1)
        wid = lax.axis_index("s") * _NC + lax.axis_index("c")
        base = wid * RB
        pltpu.sync_copy(x_hbm.at[pl.ds(base, RB)], idx_v)
        for i in range(4):
            pltpu.make_async_copy(
                table_hbm.at[_idx(idx_v, i >> 1, i & 1)], bufs[i],
                sems[i]).start()

        def group_body(g, carry):
            r0 = 2 * g
            accs = []
            for i in range(4):
                row = r0 + (i >> 1)
                half = i & 1
                pltpu.make_async_copy(
                    table_hbm.at[_idx(idx_v, row, half)], bufs[i],
                    sems[i]).wait()
                accs.append(_accum(bufs[i], lens[i]))
                nrow = row + 2

                @pl.when(nrow < RB)
                def _():
                    pltpu.make_async_copy(
                        table_hbm.at[_idx(idx_v, nrow, half)], bufs[i],
                        sems[i]).start()

            for k in range(2 * HP):
                out_v[r0, pl.ds(16 * k, 16)] = accs[0][k] + accs[1][k]
                out_v[r0 + 1, pl.ds(16 * k, 16)] = accs[2][k] + accs[3][k]
            return carry

        lax.fori_loop(0, RB // 2, group_body, 0)
        pltpu.sync_copy(out_v, sums_hbm.at[pl.ds(base, RB)])

    return sc_pool


def kernel(x, length, emb_weight, proj_W, proj_b):
    B, L = x.shape
    H, D = proj_W.shape
    xi = x.astype(jnp.int32) * 2
    table = _project_table(emb_weight, proj_W.T)
    sums = _make_sc_pool(B, L, H)(xi, table)
    return _finalize(sums, length, proj_b)


# speedup vs baseline: 8.9278x; 1.0193x over previous
import functools

import jax
import jax.numpy as jnp
from jax import lax
from jax.experimental import pallas as pl
from jax.experimental.pallas import tpu as pltpu
from jax.experimental.pallas import tpu_sc as plsc

_NC = 2
_NS = 16
_NW = _NC * _NS


def _proj_body(a_ref, blo_ref, bhi_ref, o_ref):
    dn = (((0,), (0,)), ((), ()))
    mlo = lax.dot_general(a_ref[...], blo_ref[...], dn,
                          preferred_element_type=jnp.float32)
    mhi = lax.dot_general(a_ref[...], bhi_ref[...], dn,
                          preferred_element_type=jnp.float32)
    lo = mlo.astype(jnp.bfloat16).astype(jnp.float32)
    hi = mhi.astype(jnp.bfloat16).astype(jnp.float32)
    lo_u = jax.lax.bitcast_convert_type(lo, jnp.uint32) >> 16
    hi_u = jax.lax.bitcast_convert_type(hi, jnp.uint32) & jnp.uint32(
        0xFFFF0000)
    w = jax.lax.bitcast_convert_type(lo_u | hi_u, jnp.int32)
    o_ref[...] = jnp.concatenate([w, jnp.zeros_like(w)], axis=1)


def _project_table(emb_weight, proj_Wt):
    V, D = emb_weight.shape
    H = proj_Wt.shape[1]
    HW = H // 2
    BM = 6400
    grid = (V + BM - 1) // BM
    return pl.pallas_call(
        _proj_body,
        grid=(grid,),
        in_specs=[
            pl.BlockSpec((D, BM), lambda i: (0, i)),
            pl.BlockSpec((D, HW), lambda i: (0, 0)),
            pl.BlockSpec((D, HW), lambda i: (0, 0)),
        ],
        out_specs=pl.BlockSpec((BM, 2 * HW), lambda i: (i, 0)),
        out_shape=jax.ShapeDtypeStruct((V, 2 * HW), jnp.int32),
    )(emb_weight.T, proj_Wt[:, :HW], proj_Wt[:, HW:]).reshape(2 * V, HW)


def _fin_body(s_ref, l_ref, b_ref, o_ref):
    inv = 1.0 / l_ref[...].astype(jnp.float32)
    o_ref[...] = s_ref[...] * inv + b_ref[...]


def _finalize(sums, length, proj_b):
    B, H = sums.shape
    return pl.pallas_call(
        _fin_body,
        in_specs=[
            pl.BlockSpec((B, H), lambda: (0, 0)),
            pl.BlockSpec((B, 1), lambda: (0, 0)),
            pl.BlockSpec((1, H), lambda: (0, 0)),
        ],
        out_specs=pl.BlockSpec((B, H), lambda: (0, 0)),
        out_shape=jax.ShapeDtypeStruct((B, H), jnp.float32),
    )(sums, length.reshape(B, 1), proj_b.reshape(1, H))


def _make_sc_pool(B, L, H):
    RB = B // _NW
    HP = H // 32
    HW = H // 2
    CL0 = 104
    CL1 = L - CL0

    def _accum(buf, n):
        hi_mask = jnp.int32(-65536)

        def jbody(j, carry):
            acc = list(carry)
            for k in range(HP):
                w = buf[j, pl.ds(16 * k, 16)]
                a = plsc.bitcast(w << 16, jnp.float32)
                b = plsc.bitcast(w & hi_mask, jnp.float32)
                acc[k] = acc[k] + a
                acc[HP + k] = acc[HP + k] + b
            return tuple(acc)
        init = tuple(jnp.zeros((16,), jnp.float32) for _ in range(2 * HP))
        return lax.fori_loop(0, n, jbody, init, unroll=4)

    def _idx(idx_v, row, half):
        if half == 0:
            return idx_v.at[row, pl.ds(0, CL0)]
        return idx_v.at[row, pl.ds(CL0, CL1)]

    NBUF = 8
    RG = NBUF // 2
    assert RB % RG == 0

    @functools.partial(
        pl.kernel,
        mesh=plsc.VectorSubcoreMesh(core_axis_name="c", subcore_axis_name="s"),
        out_type=jax.ShapeDtypeStruct((B, H), jnp.float32),
        compiler_params=pltpu.CompilerParams(
            needs_layout_passes=False, use_tc_tiling_on_sc=False),
        scratch_types=(
            [pltpu.VMEM((RB, L), jnp.int32)]
            + [pltpu.VMEM((CL0 if i % 2 == 0 else CL1, HW), jnp.int32)
               for i in range(NBUF)]
            + [pltpu.VMEM((RB, H), jnp.float32)]
            + [pltpu.SemaphoreType.DMA for _ in range(NBUF)]
        ),
    )
    def sc_pool(x_hbm, table_hbm, sums_hbm, idx_v, *rest):
        bufs = rest[:NBUF]
        out_v = rest[NBUF]
        sems = rest[NBUF + 1:]
        lens = tuple(CL0 if i % 2 == 0 else CL1 for i in range(NBUF))
        wid = lax.axis_index("s") * _NC + lax.axis_index("c")
        base = wid * RB
        pltpu.sync_copy(x_hbm.at[pl.ds(base, RB)], idx_v)
        for i in range(NBUF):
            pltpu.make_async_copy(
                table_hbm.at[_idx(idx_v, i >> 1, i & 1)], bufs[i],
                sems[i]).start()

        def group_body(g, carry):
            r0 = RG * g
            accs = []
            for i in range(NBUF):
                row = r0 + (i >> 1)
                half = i & 1
                pltpu.make_async_copy(
                    table_hbm.at[_idx(idx_v, row, half)], bufs[i],
                    sems[i]).wait()
                accs.append(_accum(bufs[i], lens[i]))
                nrow = row + RG

                @pl.when(nrow < RB)
                def _():
                    pltpu.make_async_copy(
                        table_hbm.at[_idx(idx_v, nrow, half)], bufs[i],
                        sems[i]).start()

            for r in range(RG):
                for k in range(2 * HP):
                    out_v[r0 + r, pl.ds(16 * k, 16)] = (
                        accs[2 * r][k] + accs[2 * r + 1][k])
            return carry

        lax.fori_loop(0, RB // RG, group_body, 0)
        pltpu.sync_copy(out_v, sums_hbm.at[pl.ds(base, RB)])

    return sc_pool


def kernel(x, length, emb_weight, proj_W, proj_b):
    B, L = x.shape
    H, D = proj_W.shape
    xi = x.astype(jnp.int32) * 2
    table = _project_table(emb_weight, proj_W.T)
    sums = _make_sc_pool(B, L, H)(xi, table)
    return _finalize(sums, length, proj_b)


# speedup vs baseline: 9.2352x; 1.0344x over previous
import functools

import jax
import jax.numpy as jnp
from jax import lax
from jax.experimental import pallas as pl
from jax.experimental.pallas import tpu as pltpu
from jax.experimental.pallas import tpu_sc as plsc

_NC = 2
_NS = 16
_NW = _NC * _NS


def _proj_body(a_ref, blo_ref, bhi_ref, o_ref):
    dn = (((0,), (0,)), ((), ()))
    mlo = lax.dot_general(a_ref[...], blo_ref[...], dn,
                          preferred_element_type=jnp.float32)
    mhi = lax.dot_general(a_ref[...], bhi_ref[...], dn,
                          preferred_element_type=jnp.float32)
    lo = mlo.astype(jnp.bfloat16).astype(jnp.float32)
    hi = mhi.astype(jnp.bfloat16).astype(jnp.float32)
    lo_u = jax.lax.bitcast_convert_type(lo, jnp.uint32) >> 16
    hi_u = jax.lax.bitcast_convert_type(hi, jnp.uint32) & jnp.uint32(
        0xFFFF0000)
    w = jax.lax.bitcast_convert_type(lo_u | hi_u, jnp.int32)
    o_ref[...] = jnp.concatenate([w, jnp.zeros_like(w)], axis=1)


def _project_table(emb_weight, proj_Wt):
    V, D = emb_weight.shape
    H = proj_Wt.shape[1]
    HW = H // 2
    BM = 12800
    grid = (V + BM - 1) // BM
    return pl.pallas_call(
        _proj_body,
        grid=(grid,),
        in_specs=[
            pl.BlockSpec((D, BM), lambda i: (0, i)),
            pl.BlockSpec((D, HW), lambda i: (0, 0)),
            pl.BlockSpec((D, HW), lambda i: (0, 0)),
        ],
        out_specs=pl.BlockSpec((BM, 2 * HW), lambda i: (i, 0)),
        out_shape=jax.ShapeDtypeStruct((V, 2 * HW), jnp.int32),
    )(emb_weight.T, proj_Wt[:, :HW], proj_Wt[:, HW:]).reshape(2 * V, HW)


def _fin_body(s_ref, l_ref, b_ref, o_ref):
    inv = 1.0 / l_ref[...].astype(jnp.float32)
    o_ref[...] = s_ref[...] * inv + b_ref[...]


def _finalize(sums, length, proj_b):
    B, H = sums.shape
    return pl.pallas_call(
        _fin_body,
        in_specs=[
            pl.BlockSpec((B, H), lambda: (0, 0)),
            pl.BlockSpec((B, 1), lambda: (0, 0)),
            pl.BlockSpec((1, H), lambda: (0, 0)),
        ],
        out_specs=pl.BlockSpec((B, H), lambda: (0, 0)),
        out_shape=jax.ShapeDtypeStruct((B, H), jnp.float32),
    )(sums, length.reshape(B, 1), proj_b.reshape(1, H))


def _make_sc_pool(B, L, H):
    RB = B // _NW
    HP = H // 32
    HW = H // 2
    CL0 = 104
    CL1 = L - CL0

    def _accum(buf, n):
        hi_mask = jnp.int32(-65536)

        def jbody(j, carry):
            acc = list(carry)
            for k in range(HP):
                w = buf[j, pl.ds(16 * k, 16)]
                a = plsc.bitcast(w << 16, jnp.float32)
                b = plsc.bitcast(w & hi_mask, jnp.float32)
                acc[k] = acc[k] + a
                acc[HP + k] = acc[HP + k] + b
            return tuple(acc)
        init = tuple(jnp.zeros((16,), jnp.float32) for _ in range(2 * HP))
        return lax.fori_loop(0, n, jbody, init, unroll=4)

    def _idx(idx_v, row, half):
        if half == 0:
            return idx_v.at[pl.ds(row * L, CL0)]
        return idx_v.at[pl.ds(row * L + CL0, CL1)]

    NBUF = 8
    RG = NBUF // 2
    assert RB % RG == 0

    @functools.partial(
        pl.kernel,
        mesh=plsc.VectorSubcoreMesh(core_axis_name="c", subcore_axis_name="s"),
        out_type=jax.ShapeDtypeStruct((B, H), jnp.float32),
        compiler_params=pltpu.CompilerParams(
            needs_layout_passes=False, use_tc_tiling_on_sc=False),
        scratch_types=(
            [pltpu.VMEM((RB * L,), jnp.int32)]
            + [pltpu.VMEM((CL0 if i % 2 == 0 else CL1, HW), jnp.int32)
               for i in range(NBUF)]
            + [pltpu.VMEM((RB, H), jnp.float32)]
            + [pltpu.SemaphoreType.DMA for _ in range(NBUF)]
        ),
    )
    def sc_pool(x_hbm, table_hbm, sums_hbm, idx_v, *rest):
        bufs = rest[:NBUF]
        out_v = rest[NBUF]
        sems = rest[NBUF + 1:]
        lens = tuple(CL0 if i % 2 == 0 else CL1 for i in range(NBUF))
        wid = lax.axis_index("s") * _NC + lax.axis_index("c")
        base = wid * RB
        pltpu.sync_copy(x_hbm.at[pl.ds(base * L, RB * L)], idx_v)

        def dbl_body(i, carry):
            w = idx_v[pl.ds(16 * i, 16)]
            idx_v[pl.ds(16 * i, 16)] = w + w
            return carry
        lax.fori_loop(0, RB * L // 16, dbl_body, 0, unroll=8)

        for i in range(NBUF):
            pltpu.make_async_copy(
                table_hbm.at[_idx(idx_v, i >> 1, i & 1)], bufs[i],
                sems[i]).start()

        def group_body(g, carry):
            r0 = RG * g
            accs = []
            for i in range(NBUF):
                row = r0 + (i >> 1)
                half = i & 1
                pltpu.make_async_copy(
                    table_hbm.at[_idx(idx_v, row, half)], bufs[i],
                    sems[i]).wait()
                accs.append(_accum(bufs[i], lens[i]))
                nrow = row + RG

                @pl.when(nrow < RB)
                def _():
                    pltpu.make_async_copy(
                        table_hbm.at[_idx(idx_v, nrow, half)], bufs[i],
                        sems[i]).start()

            for r in range(RG):
                for k in range(2 * HP):
                    out_v[r0 + r, pl.ds(16 * k, 16)] = (
                        accs[2 * r][k] + accs[2 * r + 1][k])
            return carry

        lax.fori_loop(0, RB // RG, group_body, 0)
        pltpu.sync_copy(out_v, sums_hbm.at[pl.ds(base, RB)])

    return sc_pool


def kernel(x, length, emb_weight, proj_W, proj_b):
    B, L = x.shape
    H, D = proj_W.shape
    xi = x.astype(jnp.int32).reshape(-1)
    table = _project_table(emb_weight, proj_W.T)
    sums = _make_sc_pool(B, L, H)(xi, table)
    return _finalize(sums, length, proj_b)


# speedup vs baseline: 10.0495x; 1.0882x over previous
import functools

import jax
import jax.numpy as jnp
from jax import lax
from jax.experimental import pallas as pl
from jax.experimental.pallas import tpu as pltpu
from jax.experimental.pallas import tpu_sc as plsc

_NC = 2
_NS = 16
_NW = _NC * _NS


def _proj_body(a_ref, blo_ref, bhi_ref, o_ref):
    dn = (((0,), (0,)), ((), ()))
    mlo = lax.dot_general(a_ref[...], blo_ref[...], dn,
                          preferred_element_type=jnp.float32)
    mhi = lax.dot_general(a_ref[...], bhi_ref[...], dn,
                          preferred_element_type=jnp.float32)
    lo = mlo.astype(jnp.bfloat16).astype(jnp.float32)
    hi = mhi.astype(jnp.bfloat16).astype(jnp.float32)
    lo_u = jax.lax.bitcast_convert_type(lo, jnp.uint32) >> 16
    hi_u = jax.lax.bitcast_convert_type(hi, jnp.uint32) & jnp.uint32(
        0xFFFF0000)
    w = jax.lax.bitcast_convert_type(lo_u | hi_u, jnp.int32)
    o_ref[...] = jnp.concatenate([w, jnp.zeros_like(w)], axis=1)


def _project_table(emb_weight, proj_Wt):
    V, D = emb_weight.shape
    H = proj_Wt.shape[1]
    HW = H // 2
    BM = 12800
    grid = (V + BM - 1) // BM
    return pl.pallas_call(
        _proj_body,
        grid=(grid,),
        in_specs=[
            pl.BlockSpec((D, BM), lambda i: (0, i)),
            pl.BlockSpec((D, HW), lambda i: (0, 0)),
            pl.BlockSpec((D, HW), lambda i: (0, 0)),
        ],
        out_specs=pl.BlockSpec((BM, 2 * HW), lambda i: (i, 0)),
        out_shape=jax.ShapeDtypeStruct((V, 2 * HW), jnp.int32),
    )(emb_weight.T, proj_Wt[:, :HW], proj_Wt[:, HW:]).reshape(2 * V, HW)


def _fin_body(s_ref, l_ref, b_ref, o_ref):
    inv = 1.0 / l_ref[...].astype(jnp.float32)
    o_ref[...] = s_ref[...] * inv + b_ref[...]


def _finalize(sums, length, proj_b):
    B, H = sums.shape
    return pl.pallas_call(
        _fin_body,
        in_specs=[
            pl.BlockSpec((B, H), lambda: (0, 0)),
            pl.BlockSpec((B, 1), lambda: (0, 0)),
            pl.BlockSpec((1, H), lambda: (0, 0)),
        ],
        out_specs=pl.BlockSpec((B, H), lambda: (0, 0)),
        out_shape=jax.ShapeDtypeStruct((B, H), jnp.float32),
    )(sums, length.reshape(B, 1), proj_b.reshape(1, H))


def _make_sc_pool(B, L, H):
    RB = B // _NW
    HP = H // 32
    HW = H // 2
    CL0 = 104
    CL1 = L - CL0

    def _accum(buf, n):
        def jbody(j, carry):
            acc = list(carry)
            for k in range(HP):
                w = buf[j, pl.ds(16 * k, 16)]
                a = plsc.bitcast(w << 16, jnp.float32)
                b = plsc.bitcast(w, jnp.float32)
                acc[k] = acc[k] + a
                acc[HP + k] = acc[HP + k] + b
            return tuple(acc)
        init = tuple(jnp.zeros((16,), jnp.float32) for _ in range(2 * HP))
        return lax.fori_loop(0, n, jbody, init, unroll=4)

    def _idx(idx_v, row, half):
        if half == 0:
            return idx_v.at[pl.ds(row * L, CL0)]
        return idx_v.at[pl.ds(row * L + CL0, CL1)]

    NBUF = 8
    RG = NBUF // 2
    assert RB % RG == 0

    @functools.partial(
        pl.kernel,
        mesh=plsc.VectorSubcoreMesh(core_axis_name="c", subcore_axis_name="s"),
        out_type=jax.ShapeDtypeStruct((B, H), jnp.float32),
        compiler_params=pltpu.CompilerParams(
            needs_layout_passes=False, use_tc_tiling_on_sc=False),
        scratch_types=(
            [pltpu.VMEM((RB * L,), jnp.int32)]
            + [pltpu.VMEM((CL0 if i % 2 == 0 else CL1, HW), jnp.int32)
               for i in range(NBUF)]
            + [pltpu.VMEM((RB, H), jnp.float32)]
            + [pltpu.SemaphoreType.DMA for _ in range(NBUF)]
        ),
    )
    def sc_pool(x_hbm, table_hbm, sums_hbm, idx_v, *rest):
        bufs = rest[:NBUF]
        out_v = rest[NBUF]
        sems = rest[NBUF + 1:]
        lens = tuple(CL0 if i % 2 == 0 else CL1 for i in range(NBUF))
        wid = lax.axis_index("s") * _NC + lax.axis_index("c")
        base = wid * RB
        pltpu.sync_copy(x_hbm.at[pl.ds(base * L, RB * L)], idx_v)

        def dbl_body(i, carry):
            w = idx_v[pl.ds(16 * i, 16)]
            idx_v[pl.ds(16 * i, 16)] = w + w
            return carry
        lax.fori_loop(0, RB * L // 16, dbl_body, 0, unroll=8)

        for i in range(NBUF):
            pltpu.make_async_copy(
                table_hbm.at[_idx(idx_v, i >> 1, i & 1)], bufs[i],
                sems[i]).start()

        def group_body(g, carry):
            r0 = RG * g
            accs = []
            for i in range(NBUF):
                row = r0 + (i >> 1)
                half = i & 1
                pltpu.make_async_copy(
                    table_hbm.at[_idx(idx_v, row, half)], bufs[i],
                    sems[i]).wait()
                accs.append(_accum(bufs[i], lens[i]))
                nrow = row + RG

                @pl.when(nrow < RB)
                def _():
                    pltpu.make_async_copy(
                        table_hbm.at[_idx(idx_v, nrow, half)], bufs[i],
                        sems[i]).start()

            for r in range(RG):
                for k in range(2 * HP):
                    out_v[r0 + r, pl.ds(16 * k, 16)] = (
                        accs[2 * r][k] + accs[2 * r + 1][k])
            return carry

        lax.fori_loop(0, RB // RG, group_body, 0)
        pltpu.sync_copy(out_v, sums_hbm.at[pl.ds(base, RB)])

    return sc_pool


def kernel(x, length, emb_weight, proj_W, proj_b):
    B, L = x.shape
    H, D = proj_W.shape
    xi = x.astype(jnp.int32).reshape(-1)
    table = _project_table(emb_weight, proj_W.T)
    sums = _make_sc_pool(B, L, H)(xi, table)
    return _finalize(sums, length, proj_b)
